# Initial kernel scaffold; baseline (speedup 1.0000x reference)
import jax
import jax.numpy as jnp
from jax.experimental import pallas as pl


def kernel(contRat, atom_description, alternatives):
    raise NotImplementedError("write your pallas kernel here")


# speedup vs baseline: 2.9937x; 2.9937x over previous
import jax
import jax.numpy as jnp
from jax import lax
from jax.experimental import pallas as pl
from jax.experimental.pallas import tpu as pltpu
from jax.experimental.pallas import tpu_sc as plsc

NC = 2
NS = 16
NW = NC * NS
L = 16

N_ATOMS = 2000000
CHUNK = 4000
NCHUNK = N_ATOMS // CHUNK
GROUPS = CHUNK // L
ITERS = (NCHUNK + NW - 1) // NW
NBINS = 64
ACC = NBINS * L


def _sc_body(ad_hbm, cr_hbm, aw_hbm, out_hbm,
             ad_v, cr_v, aw_v,
             mc0, mc1, mc2, sc0, sc1, sc2, ct0, ct1, ct2, res_v):
    accs = (mc0, mc1, mc2, sc0, sc1, sc2, ct0, ct1, ct2)
    cid = lax.axis_index("c")
    sid = lax.axis_index("s")
    wid = sid * NC + cid
    lane = lax.iota(jnp.int32, L)
    lane_off = lane * NBINS
    zeros = jnp.zeros((L,), jnp.float32)
    ones = jnp.ones((L,), jnp.float32)

    for a in accs:
        for q in range(ACC // L):
            a[pl.ds(q * L, L)] = zeros

    def chunk_body(i, carry):
        c = wid + i * NW

        @pl.when(c < NCHUNK)
        def _():
            pltpu.sync_copy(ad_hbm.at[pl.ds(c * (CHUNK * 5), CHUNK * 5)], ad_v)
            pltpu.sync_copy(cr_hbm.at[pl.ds(c * (CHUNK * 3), CHUNK * 3)], cr_v)
            pltpu.sync_copy(
                aw_hbm.at[pl.ds(c * (CHUNK * 3 // 4), CHUNK * 3 // 4)], aw_v)

            def group_body(g, jc):
                j5, j3 = jc
                a0 = plsc.load_gather(ad_v, [j5])
                a1 = plsc.load_gather(ad_v, [j5 + 1])
                a2 = plsc.load_gather(ad_v, [j5 + 2])
                a3 = plsc.load_gather(ad_v, [j5 + 3])
                binv = (a3 * 4 + a2) * 4 + a1
                slot = lane_off + binv
                bb = a0 < 2
                nbb = a0 >= 2
                for alt in range(3):
                    idxc = j3 if alt == 0 else j3 + alt
                    cont = plsc.load_gather(cr_v, [idxc])
                    w = plsc.load_gather(
                        aw_v, [lax.shift_right_logical(idxc, 2)])
                    sh = (idxc & 3) * 8
                    alive = (lax.shift_right_logical(w, sh) & 1) == 1
                    m_mc = alive & bb
                    m_sc = alive & nbb
                    plsc.addupdate_scatter(accs[alt], [slot], cont, mask=m_mc)
                    plsc.addupdate_scatter(accs[3 + alt], [slot], cont,
                                           mask=m_sc)
                    plsc.addupdate_scatter(accs[6 + alt], [slot], ones,
                                           mask=m_mc)
                return (j5 + 5 * L, j3 + 3 * L)

            lax.fori_loop(0, GROUPS, group_body, (lane * 5, lane * 3))
        return carry

    lax.fori_loop(0, ITERS, chunk_body, 0)

    for k in range(9):
        a = accs[k]
        for q in range(NBINS // L):
            s = a[pl.ds(q * L, L)]
            for r in range(1, L):
                s = s + a[pl.ds(r * NBINS + q * L, L)]
            res_v[pl.ds(k * NBINS + q * L, L)] = s
    pltpu.sync_copy(res_v, out_hbm.at[wid])


def _combine_body(p_ref, mc_ref, sc_ref):
    s = jnp.sum(p_ref[...], axis=0)
    mc = s[0:3]
    sc = s[3:6]
    written = s[6:9] > 0.0
    mc_ref[...] = jnp.clip(jnp.where(written, (mc - 2.0) / 38.0, mc), 0.0, 1.0)
    sc_ref[...] = jnp.clip(jnp.where(written, (sc - 5.0) / 95.0, sc), 0.0, 1.0)


def kernel(contRat, atom_description, alternatives):
    ad_flat = atom_description.astype(jnp.int32).reshape(-1)
    cr_flat = contRat.reshape(-1)
    aw = lax.bitcast_convert_type(
        alternatives.reshape(-1, 4).astype(jnp.uint8), jnp.int32)

    mesh = plsc.VectorSubcoreMesh(core_axis_name="c", subcore_axis_name="s")
    scratch = [
        pltpu.VMEM((CHUNK * 5,), jnp.int32),
        pltpu.VMEM((CHUNK * 3,), jnp.float32),
        pltpu.VMEM((CHUNK * 3 // 4,), jnp.int32),
    ] + [pltpu.VMEM((ACC,), jnp.float32) for _ in range(9)] + [
        pltpu.VMEM((9 * NBINS,), jnp.float32),
    ]
    partials = pl.kernel(
        _sc_body,
        out_type=jax.ShapeDtypeStruct((NW, 9 * NBINS), jnp.float32),
        mesh=mesh,
        scratch_types=scratch,
        compiler_params=pltpu.CompilerParams(needs_layout_passes=False),
    )(ad_flat, cr_flat, aw)

    mcn, scn = pl.pallas_call(
        _combine_body,
        out_shape=[jax.ShapeDtypeStruct((3, NBINS), jnp.float32),
                   jax.ShapeDtypeStruct((3, NBINS), jnp.float32)],
    )(partials.reshape(NW, 9, NBINS))
    rsaMC = mcn.T.reshape(4, 4, 4, 3)
    rsaSC = scn.T.reshape(4, 4, 4, 3)
    return rsaMC, rsaSC
